# baseline (speedup 1.0000x reference)
import functools

import jax
import jax.numpy as jnp
from jax import lax
from jax.experimental import pallas as pl
from jax.experimental.pallas import tpu as pltpu
from jax.experimental.pallas import tpu_sc as plsc

N_CLS = 1000
SEQ_LEN = 77
N_CTX = 16
DIM = 512

NUM_CORES = 2
NUM_SUBCORES = 16
NUM_WORKERS = NUM_CORES * NUM_SUBCORES
CLS_PER_WORKER = 32


def _splice_body(emb_flat, ctx_hbm, prefix_hbm, out_flat, ctx_v, pref_v, buf_v):
    wid = lax.axis_index("s") * NUM_CORES + lax.axis_index("c")
    base_cls = wid * CLS_PER_WORKER
    n_cls_w = jnp.minimum(CLS_PER_WORKER, N_CLS - base_cls)

    pltpu.sync_copy(ctx_hbm, ctx_v)
    pltpu.sync_copy(prefix_hbm.at[pl.ds(base_cls, CLS_PER_WORKER)], pref_v)
    pref_chunks = [pref_v[pl.ds(c * 16, 16)] for c in range(CLS_PER_WORKER // 16)]

    for k in range(CLS_PER_WORKER):
        i = base_cls + k
        p = pref_chunks[k // 16][k % 16]
        row0 = i * SEQ_LEN

        @pl.when(i < N_CLS)
        def _copy_class(p=p, row0=row0):
            pltpu.sync_copy(emb_flat.at[pl.ds(row0, SEQ_LEN)], buf_v)
            pltpu.sync_copy(buf_v, out_flat.at[pl.ds(row0, SEQ_LEN)])
            pltpu.sync_copy(ctx_v, out_flat.at[pl.ds(row0 + p, N_CTX)])


@jax.jit
def kernel(origin_text_embedding, ctx, prefix_index):
    emb_flat = origin_text_embedding.reshape(N_CLS * SEQ_LEN, DIM)
    prefix_pad = jnp.pad(
        prefix_index, (0, NUM_WORKERS * CLS_PER_WORKER - N_CLS))

    mesh = plsc.VectorSubcoreMesh(core_axis_name="c", subcore_axis_name="s")
    out_flat = pl.kernel(
        _splice_body,
        out_type=jax.ShapeDtypeStruct((N_CLS * SEQ_LEN, DIM), jnp.float32),
        mesh=mesh,
        scratch_types=[
            pltpu.VMEM((N_CTX, DIM), jnp.float32),
            pltpu.VMEM((CLS_PER_WORKER,), jnp.int32),
            pltpu.VMEM((SEQ_LEN, DIM), jnp.float32),
        ],
        compiler_params=pltpu.CompilerParams(use_tc_tiling_on_sc=False),
    )(emb_flat, ctx, prefix_pad)
    return out_flat.reshape(N_CLS, SEQ_LEN, DIM)
